# baseline (speedup 1.0000x reference)
import jax
import jax.numpy as jnp
from jax import lax
from jax.experimental import pallas as pl
from jax.experimental.pallas import tpu as pltpu
from jax.experimental.pallas import tpu_sc as plsc

VOCAB = 1000000
DIM = 64
BATCH = 16384

NUM_CORES = 2
NUM_SUBCORES = 16
NUM_WORKERS = NUM_CORES * NUM_SUBCORES
BPW = BATCH // NUM_WORKERS
IDX_CHUNK = 128
HALF = BPW // 2
LANES = 16
PAIR = 2 * DIM


def _sc_kernel_body(idx0_hbm, idx1_hbm, labels_hbm, t0_hbm, t1_hbm,
                    out_hbm, idx0_v, idx1_v, idxh0_v, idxh1_v, lab_v,
                    rows0_v, rows1_v, part_v, sem):
    wid = lax.axis_index("s") * NUM_CORES + lax.axis_index("c")
    base = wid * BPW

    pltpu.sync_copy(idx0_hbm.at[pl.ds(base, BPW)], idx0_v)
    pltpu.sync_copy(idx1_hbm.at[pl.ds(base, BPW)], idx1_v)
    pltpu.sync_copy(labels_hbm.at[pl.ds(base, BPW)], lab_v)

    def shift_step(i, carry):
        s = pl.ds(i * LANES, LANES)
        idxh0_v[s] = lax.shift_right_logical(idx0_v[s], 1)
        idxh1_v[s] = lax.shift_right_logical(idx1_v[s], 1)
        return carry

    lax.fori_loop(0, BPW // LANES, shift_step, 0)

    lane = lax.broadcasted_iota(jnp.int32, (LANES,), 0)
    loss_acc = jnp.zeros((LANES,), jnp.float32)

    for h in range(BPW // HALF):
        copies = []
        for c in range(HALF // IDX_CHUNK):
            j = h * (HALF // IDX_CHUNK) + c
            src = pl.ds(j * IDX_CHUNK, IDX_CHUNK)
            dst = pl.ds(c * IDX_CHUNK, IDX_CHUNK)
            copies.append(pltpu.async_copy(
                t0_hbm.at[idxh0_v.at[src]], rows0_v.at[dst], sem))
            copies.append(pltpu.async_copy(
                t1_hbm.at[idxh1_v.at[src]], rows1_v.at[dst], sem))
        for cp in copies:
            cp.wait()

        def group_step(g, acc_in, h=h):
            rel = lane + g * LANES
            s_abs = pl.ds(h * HALF + g * LANES, LANES)
            p0 = (idx0_v[s_abs] & 1) * DIM
            p1 = (idx1_v[s_abs] & 1) * DIM

            def d_step(d, acc):
                v0 = plsc.load_gather(rows0_v, [rel, p0 + d])
                v1 = plsc.load_gather(rows1_v, [rel, p1 + d])
                return acc + v0 * v1

            pred = lax.fori_loop(0, DIM, d_step,
                                 jnp.zeros((LANES,), jnp.float32))
            sig = 1.0 / (1.0 + jnp.exp(-pred))
            diff = sig - lab_v[s_abs]
            return acc_in + diff * diff

        loss_acc = lax.fori_loop(0, HALF // LANES, group_step, loss_acc)

    part_v[...] = loss_acc
    pltpu.sync_copy(part_v, out_hbm.at[pl.ds(wid * LANES, LANES)])


@jax.jit
def _run(idx0, idx1, labels, t0, t1):
    mesh = plsc.VectorSubcoreMesh(core_axis_name="c", subcore_axis_name="s")
    partials = pl.kernel(
        _sc_kernel_body,
        out_type=jax.ShapeDtypeStruct((NUM_WORKERS * LANES,), jnp.float32),
        mesh=mesh,
        compiler_params=pltpu.CompilerParams(
            needs_layout_passes=False, use_tc_tiling_on_sc=True),
        scratch_types=[
            pltpu.VMEM((BPW,), jnp.int32),
            pltpu.VMEM((BPW,), jnp.int32),
            pltpu.VMEM((BPW,), jnp.int32),
            pltpu.VMEM((BPW,), jnp.int32),
            pltpu.VMEM((BPW,), jnp.float32),
            pltpu.VMEM((HALF, PAIR), jnp.float32),
            pltpu.VMEM((HALF, PAIR), jnp.float32),
            pltpu.VMEM((LANES,), jnp.float32),
            pltpu.SemaphoreType.DMA,
        ],
    )(idx0, idx1, labels, t0, t1)
    return jnp.sum(partials) * (1.0 / BATCH)


def kernel(indices_f0, indices_f1, labels, emb_table_0, emb_table_1):
    idx0 = indices_f0.astype(jnp.int32)
    idx1 = indices_f1.astype(jnp.int32)
    t0 = emb_table_0.reshape(VOCAB // 2, PAIR)
    t1 = emb_table_1.reshape(VOCAB // 2, PAIR)
    return _run(idx0, idx1, labels, t0, t1)


# speedup vs baseline: 2.7078x; 2.7078x over previous
import jax
import jax.numpy as jnp
from jax import lax
from jax.experimental import pallas as pl
from jax.experimental.pallas import tpu as pltpu
from jax.experimental.pallas import tpu_sc as plsc

VOCAB = 1000000
DIM = 64
BATCH = 16384

NUM_CORES = 2
NUM_SUBCORES = 16
NUM_WORKERS = NUM_CORES * NUM_SUBCORES
TD_PER_CORE = 4
BPS = BATCH // NUM_SUBCORES
BPW = BATCH // NUM_WORKERS
IDX_CHUNK = 128
NCHUNK = BPS // IDX_CHUNK
LANES = 16


def _dot_kernel_body(idx0_hbm, idx1_hbm, t0_hbm, t1_hbm, parts_hbm,
                     sp0, sp1, idx0_v, idx1_v, v0_v, v1_v, acc_v, sem):
    cid = lax.axis_index("c")
    sid = lax.axis_index("s")

    pltpu.sync_copy(idx0_hbm.at[pl.ds(sid * NCHUNK, NCHUNK)], idx0_v)
    pltpu.sync_copy(idx1_hbm.at[pl.ds(sid * NCHUNK, NCHUNK)], idx1_v)

    def zero_step(i, carry):
        acc_v[pl.ds(i * LANES, LANES)] = jnp.zeros((LANES,), jnp.float32)
        return carry

    lax.fori_loop(0, BPS // LANES, zero_step, 0)

    def td_step(tdl, carry):
        td = cid * TD_PER_CORE + tdl
        for sd in range(8):
            @pl.when(sid == 0)
            def _():
                pltpu.sync_copy(t0_hbm.at[td, sd], sp0)

            @pl.when(sid == 1)
            def _():
                pltpu.sync_copy(t1_hbm.at[td, sd], sp1)

            plsc.subcore_barrier()

            copies = []
            for c in range(NCHUNK):
                dst = pl.ds(c * IDX_CHUNK, IDX_CHUNK)
                copies.append(pltpu.async_copy(
                    sp0.at[idx0_v.at[c]], v0_v.at[dst], sem))
                copies.append(pltpu.async_copy(
                    sp1.at[idx1_v.at[c]], v1_v.at[dst], sem))
            for cp in copies:
                cp.wait()

            plsc.subcore_barrier()

            def acc_step(i, c2):
                s = pl.ds(i * LANES, LANES)
                acc_v[s] = acc_v[s] + v0_v[s] * v1_v[s]
                return c2

            lax.fori_loop(0, BPS // LANES, acc_step, 0)
        return carry

    lax.fori_loop(0, TD_PER_CORE, td_step, 0)

    pltpu.sync_copy(
        acc_v, parts_hbm.at[pl.ds(cid * BATCH + sid * BPS, BPS)])


def _loss_kernel_body(parts_hbm, labels_hbm, out_hbm, p0_v, p1_v, lab_v,
                      part_v):
    wid = lax.axis_index("s") * NUM_CORES + lax.axis_index("c")
    base = wid * BPW

    pltpu.sync_copy(parts_hbm.at[pl.ds(base, BPW)], p0_v)
    pltpu.sync_copy(parts_hbm.at[pl.ds(BATCH + base, BPW)], p1_v)
    pltpu.sync_copy(labels_hbm.at[pl.ds(base, BPW)], lab_v)

    def loss_step(g, loss_acc):
        s = pl.ds(g * LANES, LANES)
        pred = p0_v[s] + p1_v[s]
        sig = 1.0 / (1.0 + jnp.exp(-pred))
        diff = sig - lab_v[s]
        return loss_acc + diff * diff

    loss_acc = lax.fori_loop(0, BPW // LANES, loss_step,
                             jnp.zeros((LANES,), jnp.float32))

    part_v[...] = loss_acc
    pltpu.sync_copy(part_v, out_hbm.at[pl.ds(wid * LANES, LANES)])


@jax.jit
def _run(idx0, idx1, labels, t0, t1):
    mesh = plsc.VectorSubcoreMesh(core_axis_name="c", subcore_axis_name="s")
    parts = pl.kernel(
        _dot_kernel_body,
        out_type=jax.ShapeDtypeStruct((NUM_CORES * BATCH,), jnp.float32),
        mesh=mesh,
        compiler_params=pltpu.CompilerParams(
            needs_layout_passes=False, use_tc_tiling_on_sc=True),
        scratch_types=[
            pltpu.VMEM_SHARED((VOCAB,), jnp.float32),
            pltpu.VMEM_SHARED((VOCAB,), jnp.float32),
            pltpu.VMEM((NCHUNK, IDX_CHUNK), jnp.int32),
            pltpu.VMEM((NCHUNK, IDX_CHUNK), jnp.int32),
            pltpu.VMEM((BPS,), jnp.float32),
            pltpu.VMEM((BPS,), jnp.float32),
            pltpu.VMEM((BPS,), jnp.float32),
            pltpu.SemaphoreType.DMA,
        ],
    )(idx0, idx1, t0, t1)

    losses = pl.kernel(
        _loss_kernel_body,
        out_type=jax.ShapeDtypeStruct((NUM_WORKERS * LANES,), jnp.float32),
        mesh=mesh,
        compiler_params=pltpu.CompilerParams(
            needs_layout_passes=False, use_tc_tiling_on_sc=True),
        scratch_types=[
            pltpu.VMEM((BPW,), jnp.float32),
            pltpu.VMEM((BPW,), jnp.float32),
            pltpu.VMEM((BPW,), jnp.float32),
            pltpu.VMEM((LANES,), jnp.float32),
        ],
    )(parts, labels)

    return jnp.sum(losses) * (1.0 / BATCH)


def kernel(indices_f0, indices_f1, labels, emb_table_0, emb_table_1):
    idx0 = indices_f0.astype(jnp.int32).reshape(BATCH // IDX_CHUNK,
                                                IDX_CHUNK)
    idx1 = indices_f1.astype(jnp.int32).reshape(BATCH // IDX_CHUNK,
                                                IDX_CHUNK)
    t0 = emb_table_0.T.reshape(8, 8, VOCAB)
    t1 = emb_table_1.T.reshape(8, 8, VOCAB)
    return _run(idx0, idx1, labels, t0, t1)
